# Initial kernel scaffold; baseline (speedup 1.0000x reference)
import jax
import jax.numpy as jnp
from jax.experimental import pallas as pl


def kernel(x, edge_index, W1_l, W1_r, b1, W2_l, W2_r, b2):
    raise NotImplementedError("write your pallas kernel here")


# speedup vs baseline: 2.9622x; 2.9622x over previous
import jax
import jax.numpy as jnp
from jax import lax
from jax.experimental import pallas as pl
from jax.experimental.pallas import tpu as pltpu
from jax.experimental.pallas import tpu_sc as plsc

N = 10000
E = 320000
F_IN = 128
HID = 128
CLS = 64

NC = 2
NS = 16
HALF = 5120
HPAD = 5376
TRASH = HALF
EPW = E // NS
CHUNK = 80
NCHUNK = EPW // CHUNK
RPZ = HPAD // NS
RPT = HALF // NS
DEGW = 16
ZROWS = 16


def _make_sc_agg(with_deg):
  mesh = plsc.VectorSubcoreMesh(core_axis_name="c", subcore_axis_name="s")
  out_type = [jax.ShapeDtypeStruct((NC, HALF, F_IN), jnp.float32)]
  scratch = [
      pltpu.VMEM((CHUNK,), jnp.int32),
      pltpu.VMEM((CHUNK,), jnp.int32),
      pltpu.VMEM((CHUNK, F_IN), jnp.float32),
      pltpu.VMEM((ZROWS, F_IN), jnp.float32),
      pltpu.SemaphoreType.DMA,
      pltpu.VMEM_SHARED((HPAD, F_IN), jnp.float32),
  ]
  if with_deg:
    out_type.append(jax.ShapeDtypeStruct((NC, HALF, DEGW), jnp.float32))
    scratch += [
        pltpu.VMEM((CHUNK, DEGW), jnp.float32),
        pltpu.VMEM((ZROWS, DEGW), jnp.float32),
        pltpu.VMEM_SHARED((HPAD, DEGW), jnp.float32),
    ]

  def body(table, src, dst, *rest):
    if with_deg:
      (out, deg_out, idx_s, idx_d, rows, zb, sem, agg_sh,
       ones_v, zbd, deg_sh) = rest
    else:
      out, idx_s, idx_d, rows, zb, sem, agg_sh = rest

    cid = lax.axis_index("c")
    sid = lax.axis_index("s")

    z0 = sid * RPZ
    for r in range(ZROWS):
      for c in range(F_IN // 16):
        zb[r, pl.ds(c * 16, 16)] = jnp.zeros((16,), jnp.float32)
    for j in range(RPZ // ZROWS):
      pltpu.sync_copy(zb, agg_sh.at[pl.ds(z0 + j * ZROWS, ZROWS)])
    if with_deg:
      for r in range(ZROWS):
        zbd[r, :] = jnp.zeros((DEGW,), jnp.float32)
      for r in range(CHUNK):
        ones_v[r, :] = jnp.ones((DEGW,), jnp.float32)
      for j in range(RPZ // ZROWS):
        pltpu.sync_copy(zbd, deg_sh.at[pl.ds(z0 + j * ZROWS, ZROWS)])
    plsc.subcore_barrier()

    base0 = sid * EPW
    lo = cid * HALF

    @pl.loop(0, NCHUNK)
    def _(i):
      b = base0 + i * CHUNK
      pltpu.sync_copy(src.at[pl.ds(b, CHUNK)], idx_s)
      pltpu.sync_copy(dst.at[pl.ds(b, CHUNK)], idx_d)
      for k in range(CHUNK // 16):
        sl = pl.ds(k * 16, 16)
        v = idx_d[sl] - lo
        ok = jnp.logical_and(v >= 0, v < HALF)
        idx_d[sl] = jnp.where(ok, v, TRASH)
      pltpu.async_copy(table.at[idx_s], rows, sem).wait()
      pltpu.sync_copy(rows, agg_sh.at[idx_d], add=True)
      if with_deg:
        pltpu.sync_copy(ones_v, deg_sh.at[idx_d], add=True)

    plsc.subcore_barrier()

    r0 = sid * RPT
    for j in range(RPT // CHUNK):
      rs = pl.ds(r0 + j * CHUNK, CHUNK)
      pltpu.sync_copy(agg_sh.at[rs], rows)
      pltpu.sync_copy(rows, out.at[cid, rs])
      if with_deg:
        pltpu.sync_copy(deg_sh.at[rs], ones_v)
        pltpu.sync_copy(ones_v, deg_out.at[cid, rs])

  return pl.kernel(body, out_type=tuple(out_type), mesh=mesh,
                   scratch_types=scratch)


_sc_agg_deg = _make_sc_agg(True)
_sc_agg = _make_sc_agg(False)


ROWS_TC = 1000


def _rdeg(degs_ref):
  ones = jnp.ones((DEGW, F_IN), jnp.float32)
  d = jnp.dot(degs_ref[...], ones,
              preferred_element_type=jnp.float32) * (1.0 / DEGW)
  return 1.0 / jnp.maximum(d, 1.0)


def _tc1_body(parts_ref, degs_ref, x_ref, wl_ref, wr_ref, b_ref, out_ref):
  agg = parts_ref[...] * _rdeg(degs_ref)
  h = (jnp.dot(agg, wl_ref[...], preferred_element_type=jnp.float32)
       + jnp.dot(x_ref[...], wr_ref[...], preferred_element_type=jnp.float32)
       + b_ref[...])
  out_ref[...] = jnp.maximum(h, 0.0)


def _tc2_body(parts_ref, degs_ref, h_ref, wl_ref, wr_ref, b_ref, out_ref):
  agg = parts_ref[...] * _rdeg(degs_ref)
  z = (jnp.dot(agg, wl_ref[...], preferred_element_type=jnp.float32)
       + jnp.dot(h_ref[...], wr_ref[...], preferred_element_type=jnp.float32)
       + b_ref[...])
  m = jnp.max(z, axis=1, keepdims=True)
  e = z - m
  out_ref[...] = e - jnp.log(jnp.sum(jnp.exp(e), axis=1, keepdims=True))


def _tc_layer(body, dout, parts, degs, feat, wl, wr, b):
  grid = (N // ROWS_TC,)
  return pl.pallas_call(
      body,
      grid=grid,
      in_specs=[
          pl.BlockSpec((ROWS_TC, F_IN), lambda i: (i, 0)),
          pl.BlockSpec((ROWS_TC, DEGW), lambda i: (i, 0)),
          pl.BlockSpec((ROWS_TC, F_IN), lambda i: (i, 0)),
          pl.BlockSpec((F_IN, dout), lambda i: (0, 0)),
          pl.BlockSpec((F_IN, dout), lambda i: (0, 0)),
          pl.BlockSpec((1, dout), lambda i: (0, 0)),
      ],
      out_specs=pl.BlockSpec((ROWS_TC, dout), lambda i: (i, 0)),
      out_shape=jax.ShapeDtypeStruct((N, dout), jnp.float32),
  )(parts, degs, feat, wl, wr, b)


def kernel(x, edge_index, W1_l, W1_r, b1, W2_l, W2_r, b2):
  src = edge_index[0]
  dst = edge_index[1]
  parts1, degp = _sc_agg_deg(x, src, dst)
  parts1 = parts1.reshape(NC * HALF, F_IN)
  degf = degp.reshape(NC * HALF, DEGW)
  h = _tc_layer(_tc1_body, HID, parts1, degf, x, W1_l, W1_r,
                b1.reshape(1, HID))
  (parts2,) = _sc_agg(h, src, dst)
  out = _tc_layer(_tc2_body, CLS, parts2.reshape(NC * HALF, F_IN), degf, h,
                  W2_l, W2_r, b2.reshape(1, CLS))
  return out


# speedup vs baseline: 5.1848x; 1.7503x over previous
import jax
import jax.numpy as jnp
from jax import lax
from jax.experimental import pallas as pl
from jax.experimental.pallas import tpu as pltpu
from jax.experimental.pallas import tpu_sc as plsc

N = 10000
E = 320000
F_IN = 128
HID = 128
CLS = 64

NC = 2
NS = 16
HALF = 5120
HPAD = 5376
TRASH = HALF
EPW = E // NS
CHUNK = 80
BLK = 800
CPB = BLK // CHUNK
NBLK = EPW // BLK
RPZ = HPAD // NS
RPT = HALF // NS
DEGW = 16
ZROWS = 16


def _make_sc_agg(with_deg):
  mesh = plsc.VectorSubcoreMesh(core_axis_name="c", subcore_axis_name="s")
  out_type = [jax.ShapeDtypeStruct((NC, HALF, F_IN), jnp.float32)]
  scratch = [
      pltpu.VMEM((BLK,), jnp.int32),
      pltpu.VMEM((BLK,), jnp.int32),
      pltpu.VMEM((CHUNK, F_IN), jnp.float32),
      pltpu.VMEM((CHUNK, F_IN), jnp.float32),
      pltpu.VMEM((ZROWS, F_IN), jnp.float32),
      pltpu.SemaphoreType.DMA,
      pltpu.SemaphoreType.DMA,
      pltpu.VMEM_SHARED((HPAD, F_IN), jnp.float32),
  ]
  if with_deg:
    out_type.append(jax.ShapeDtypeStruct((NC, HALF, DEGW), jnp.float32))
    scratch += [
        pltpu.VMEM((CHUNK, DEGW), jnp.float32),
        pltpu.VMEM((ZROWS, DEGW), jnp.float32),
        pltpu.VMEM_SHARED((HPAD, DEGW), jnp.float32),
    ]

  def body(table, src, dst, *rest):
    if with_deg:
      (out, deg_out, idx_s, idx_d, rows0, rows1, zb, sem0, sem1, agg_sh,
       ones_v, zbd, deg_sh) = rest
    else:
      out, idx_s, idx_d, rows0, rows1, zb, sem0, sem1, agg_sh = rest
    rows = rows0

    cid = lax.axis_index("c")
    sid = lax.axis_index("s")

    z0 = sid * RPZ
    for r in range(ZROWS):
      for c in range(F_IN // 16):
        zb[r, pl.ds(c * 16, 16)] = jnp.zeros((16,), jnp.float32)
    for j in range(RPZ // ZROWS):
      pltpu.sync_copy(zb, agg_sh.at[pl.ds(z0 + j * ZROWS, ZROWS)])
    if with_deg:
      for r in range(ZROWS):
        zbd[r, :] = jnp.zeros((DEGW,), jnp.float32)
      for r in range(CHUNK):
        ones_v[r, :] = jnp.ones((DEGW,), jnp.float32)
      for j in range(RPZ // ZROWS):
        pltpu.sync_copy(zbd, deg_sh.at[pl.ds(z0 + j * ZROWS, ZROWS)])
    plsc.subcore_barrier()

    base0 = sid * EPW
    lo = cid * HALF

    bufs = (rows0, rows1)
    sems = (sem0, sem1)

    @pl.loop(0, NBLK)
    def _(blk):
      e0 = base0 + blk * BLK
      pltpu.sync_copy(src.at[pl.ds(e0, BLK)], idx_s)
      pltpu.sync_copy(dst.at[pl.ds(e0, BLK)], idx_d)
      for k in range(BLK // 16):
        sl = pl.ds(k * 16, 16)
        v = idx_d[sl] - lo
        ok = jnp.logical_and(v >= 0, v < HALF)
        idx_d[sl] = jnp.where(ok, v, TRASH)

      def gather(j):
        return pltpu.async_copy(
            table.at[idx_s.at[pl.ds(j * CHUNK, CHUNK)]],
            bufs[j % 2], sems[j % 2])

      cp = gather(0)
      for j in range(CPB):
        nxt = gather(j + 1) if j + 1 < CPB else None
        cp.wait()
        dv = idx_d.at[pl.ds(j * CHUNK, CHUNK)]
        pltpu.sync_copy(bufs[j % 2], agg_sh.at[dv], add=True)
        if with_deg:
          pltpu.sync_copy(ones_v, deg_sh.at[dv], add=True)
        cp = nxt

    plsc.subcore_barrier()

    r0 = sid * RPT
    for j in range(RPT // CHUNK):
      rs = pl.ds(r0 + j * CHUNK, CHUNK)
      pltpu.sync_copy(agg_sh.at[rs], rows)
      pltpu.sync_copy(rows, out.at[cid, rs])
      if with_deg:
        pltpu.sync_copy(deg_sh.at[rs], ones_v)
        pltpu.sync_copy(ones_v, deg_out.at[cid, rs])

  return pl.kernel(body, out_type=tuple(out_type), mesh=mesh,
                   scratch_types=scratch)


_sc_agg_deg = _make_sc_agg(True)
_sc_agg = _make_sc_agg(False)


ROWS_TC = 1000


def _rdeg(degs_ref):
  ones = jnp.ones((DEGW, F_IN), jnp.float32)
  d = jnp.dot(degs_ref[...], ones,
              preferred_element_type=jnp.float32) * (1.0 / DEGW)
  return 1.0 / jnp.maximum(d, 1.0)


def _tc1_body(parts_ref, degs_ref, x_ref, wl_ref, wr_ref, b_ref, out_ref):
  agg = parts_ref[...] * _rdeg(degs_ref)
  h = (jnp.dot(agg, wl_ref[...], preferred_element_type=jnp.float32)
       + jnp.dot(x_ref[...], wr_ref[...], preferred_element_type=jnp.float32)
       + b_ref[...])
  out_ref[...] = jnp.maximum(h, 0.0)


def _tc2_body(parts_ref, degs_ref, h_ref, wl_ref, wr_ref, b_ref, out_ref):
  agg = parts_ref[...] * _rdeg(degs_ref)
  z = (jnp.dot(agg, wl_ref[...], preferred_element_type=jnp.float32)
       + jnp.dot(h_ref[...], wr_ref[...], preferred_element_type=jnp.float32)
       + b_ref[...])
  m = jnp.max(z, axis=1, keepdims=True)
  e = z - m
  out_ref[...] = e - jnp.log(jnp.sum(jnp.exp(e), axis=1, keepdims=True))


def _tc_layer(body, dout, parts, degs, feat, wl, wr, b):
  grid = (N // ROWS_TC,)
  return pl.pallas_call(
      body,
      grid=grid,
      in_specs=[
          pl.BlockSpec((ROWS_TC, F_IN), lambda i: (i, 0)),
          pl.BlockSpec((ROWS_TC, DEGW), lambda i: (i, 0)),
          pl.BlockSpec((ROWS_TC, F_IN), lambda i: (i, 0)),
          pl.BlockSpec((F_IN, dout), lambda i: (0, 0)),
          pl.BlockSpec((F_IN, dout), lambda i: (0, 0)),
          pl.BlockSpec((1, dout), lambda i: (0, 0)),
      ],
      out_specs=pl.BlockSpec((ROWS_TC, dout), lambda i: (i, 0)),
      out_shape=jax.ShapeDtypeStruct((N, dout), jnp.float32),
  )(parts, degs, feat, wl, wr, b)


def kernel(x, edge_index, W1_l, W1_r, b1, W2_l, W2_r, b2):
  src = edge_index[0]
  dst = edge_index[1]
  parts1, degp = _sc_agg_deg(x, src, dst)
  parts1 = parts1.reshape(NC * HALF, F_IN)
  degf = degp.reshape(NC * HALF, DEGW)
  h = _tc_layer(_tc1_body, HID, parts1, degf, x, W1_l, W1_r,
                b1.reshape(1, HID))
  (parts2,) = _sc_agg(h, src, dst)
  out = _tc_layer(_tc2_body, CLS, parts2.reshape(NC * HALF, F_IN), degf, h,
                  W2_l, W2_r, b2.reshape(1, CLS))
  return out


# speedup vs baseline: 6.3484x; 1.2244x over previous
import jax
import jax.numpy as jnp
from jax import lax
from jax.experimental import pallas as pl
from jax.experimental.pallas import tpu as pltpu
from jax.experimental.pallas import tpu_sc as plsc

N = 10000
E = 320000
F_IN = 128
HID = 128
CLS = 64

NC = 2
NS = 16
HALF = 5120
HPAD = 5376
TRASH = HALF
EPW = E // NS
CHUNK = 80
BLK = 800
CPB = BLK // CHUNK
NBLK = EPW // BLK
RPZ = HPAD // NS
RPT = HALF // NS
DEGW = 16
ZROWS = 16


def _make_sc_agg(with_deg):
  mesh = plsc.VectorSubcoreMesh(core_axis_name="c", subcore_axis_name="s")
  out_type = [jax.ShapeDtypeStruct((NC, HALF, F_IN), jnp.float32)]
  scratch = [
      pltpu.VMEM((BLK,), jnp.int32),
      pltpu.VMEM((BLK,), jnp.int32),
      pltpu.VMEM((CHUNK, F_IN), jnp.float32),
      pltpu.VMEM((CHUNK, F_IN), jnp.float32),
      pltpu.VMEM((ZROWS, F_IN), jnp.float32),
      pltpu.SemaphoreType.DMA,
      pltpu.SemaphoreType.DMA,
      pltpu.VMEM_SHARED((HPAD, F_IN), jnp.float32),
  ]
  if with_deg:
    out_type.append(jax.ShapeDtypeStruct((NC, HALF, DEGW), jnp.float32))
    scratch += [
        pltpu.VMEM((CHUNK, DEGW), jnp.float32),
        pltpu.VMEM((ZROWS, DEGW), jnp.float32),
        pltpu.VMEM_SHARED((HPAD, DEGW), jnp.float32),
    ]

  def body(table, src, dst, *rest):
    if with_deg:
      (out, deg_out, idx_s, idx_d, rows0, rows1, zb, sem0, sem1, agg_sh,
       ones_v, zbd, deg_sh) = rest
    else:
      out, idx_s, idx_d, rows0, rows1, zb, sem0, sem1, agg_sh = rest
    rows = rows0

    cid = lax.axis_index("c")
    sid = lax.axis_index("s")

    z0 = sid * RPZ
    for r in range(ZROWS):
      for c in range(F_IN // 16):
        zb[r, pl.ds(c * 16, 16)] = jnp.zeros((16,), jnp.float32)
    for j in range(RPZ // ZROWS):
      pltpu.sync_copy(zb, agg_sh.at[pl.ds(z0 + j * ZROWS, ZROWS)])
    if with_deg:
      for r in range(ZROWS):
        zbd[r, :] = jnp.zeros((DEGW,), jnp.float32)
      for r in range(CHUNK):
        ones_v[r, :] = jnp.ones((DEGW,), jnp.float32)
      for j in range(RPZ // ZROWS):
        pltpu.sync_copy(zbd, deg_sh.at[pl.ds(z0 + j * ZROWS, ZROWS)])
    plsc.subcore_barrier()

    base0 = sid * EPW
    lo = cid * HALF

    bufs = (rows0, rows1)
    sems = (sem0, sem1)
    trash = HALF + sid * 16 + jax.lax.iota(jnp.int32, 16)

    @pl.loop(0, NBLK)
    def _(blk):
      e0 = base0 + blk * BLK
      pltpu.sync_copy(src.at[pl.ds(e0, BLK)], idx_s)
      pltpu.sync_copy(dst.at[pl.ds(e0, BLK)], idx_d)
      for k in range(BLK // 16):
        sl = pl.ds(k * 16, 16)
        v = idx_d[sl] - lo
        ok = jnp.logical_and(v >= 0, v < HALF)
        idx_d[sl] = jnp.where(ok, v, trash)

      def gather(j):
        return pltpu.async_copy(
            table.at[idx_s.at[pl.ds(j * CHUNK, CHUNK)]],
            bufs[j % 2], sems[j % 2])

      cp = gather(0)
      for j in range(CPB):
        nxt = gather(j + 1) if j + 1 < CPB else None
        cp.wait()
        dv = idx_d.at[pl.ds(j * CHUNK, CHUNK)]
        pltpu.sync_copy(bufs[j % 2], agg_sh.at[dv], add=True)
        if with_deg:
          pltpu.sync_copy(ones_v, deg_sh.at[dv], add=True)
        cp = nxt

    plsc.subcore_barrier()

    r0 = sid * RPT
    copc = 80
    for j in range(RPT // copc):
      rs = pl.ds(r0 + j * copc, copc)
      pltpu.sync_copy(agg_sh.at[rs], rows.at[pl.ds(0, copc)])
      pltpu.sync_copy(rows.at[pl.ds(0, copc)], out.at[cid, rs])
      if with_deg:
        pltpu.sync_copy(deg_sh.at[rs], ones_v.at[pl.ds(0, copc)])
        pltpu.sync_copy(ones_v.at[pl.ds(0, copc)], deg_out.at[cid, rs])

  return pl.kernel(body, out_type=tuple(out_type), mesh=mesh,
                   scratch_types=scratch)


_sc_agg_deg = _make_sc_agg(True)
_sc_agg = _make_sc_agg(False)


ROWS_TC = 1000


def _rdeg(degs_ref):
  ones = jnp.ones((DEGW, F_IN), jnp.float32)
  d = jnp.dot(degs_ref[...], ones,
              preferred_element_type=jnp.float32) * (1.0 / DEGW)
  return 1.0 / jnp.maximum(d, 1.0)


def _tc1_body(parts_ref, degs_ref, x_ref, wl_ref, wr_ref, b_ref, out_ref):
  agg = parts_ref[...] * _rdeg(degs_ref)
  h = (jnp.dot(agg, wl_ref[...], preferred_element_type=jnp.float32)
       + jnp.dot(x_ref[...], wr_ref[...], preferred_element_type=jnp.float32)
       + b_ref[...])
  out_ref[...] = jnp.maximum(h, 0.0)


def _tc2_body(parts_ref, degs_ref, h_ref, wl_ref, wr_ref, b_ref, out_ref):
  agg = parts_ref[...] * _rdeg(degs_ref)
  z = (jnp.dot(agg, wl_ref[...], preferred_element_type=jnp.float32)
       + jnp.dot(h_ref[...], wr_ref[...], preferred_element_type=jnp.float32)
       + b_ref[...])
  m = jnp.max(z, axis=1, keepdims=True)
  e = z - m
  out_ref[...] = e - jnp.log(jnp.sum(jnp.exp(e), axis=1, keepdims=True))


def _tc_layer(body, dout, parts, degs, feat, wl, wr, b):
  grid = (N // ROWS_TC,)
  return pl.pallas_call(
      body,
      grid=grid,
      in_specs=[
          pl.BlockSpec((ROWS_TC, F_IN), lambda i: (i, 0)),
          pl.BlockSpec((ROWS_TC, DEGW), lambda i: (i, 0)),
          pl.BlockSpec((ROWS_TC, F_IN), lambda i: (i, 0)),
          pl.BlockSpec((F_IN, dout), lambda i: (0, 0)),
          pl.BlockSpec((F_IN, dout), lambda i: (0, 0)),
          pl.BlockSpec((1, dout), lambda i: (0, 0)),
      ],
      out_specs=pl.BlockSpec((ROWS_TC, dout), lambda i: (i, 0)),
      out_shape=jax.ShapeDtypeStruct((N, dout), jnp.float32),
  )(parts, degs, feat, wl, wr, b)


def kernel(x, edge_index, W1_l, W1_r, b1, W2_l, W2_r, b2):
  src = edge_index[0]
  dst = edge_index[1]
  parts1, degp = _sc_agg_deg(x, src, dst)
  parts1 = parts1.reshape(NC * HALF, F_IN)
  degf = degp.reshape(NC * HALF, DEGW)
  h = _tc_layer(_tc1_body, HID, parts1, degf, x, W1_l, W1_r,
                b1.reshape(1, HID))
  (parts2,) = _sc_agg(h, src, dst)
  out = _tc_layer(_tc2_body, CLS, parts2.reshape(NC * HALF, F_IN), degf, h,
                  W2_l, W2_r, b2.reshape(1, CLS))
  return out


# speedup vs baseline: 6.9881x; 1.1008x over previous
import jax
import jax.numpy as jnp
from jax import lax
from jax.experimental import pallas as pl
from jax.experimental.pallas import tpu as pltpu
from jax.experimental.pallas import tpu_sc as plsc

N = 10000
E = 320000
F_IN = 128
HID = 128
CLS = 64

NC = 2
NS = 16
HALF = 5120
HPAD = 5376
TRASH = HALF
EPW = E // NS
CHUNK = 80
BLK = 2000
CPB = BLK // CHUNK
NBLK = EPW // BLK
RPZ = HPAD // NS
RPT = HALF // NS
DEGW = 16
ZROWS = 16


def _make_sc_agg(with_deg):
  mesh = plsc.VectorSubcoreMesh(core_axis_name="c", subcore_axis_name="s")
  out_type = [jax.ShapeDtypeStruct((NC, HALF, F_IN), jnp.float32)]
  scratch = [
      pltpu.VMEM((BLK,), jnp.int32),
      pltpu.VMEM((BLK,), jnp.int32),
      pltpu.VMEM((CHUNK, F_IN), jnp.float32),
      pltpu.VMEM((CHUNK, F_IN), jnp.float32),
      pltpu.VMEM((ZROWS, F_IN), jnp.float32),
      pltpu.SemaphoreType.DMA,
      pltpu.SemaphoreType.DMA,
      pltpu.VMEM_SHARED((HPAD, F_IN), jnp.float32),
  ]
  if with_deg:
    out_type.append(jax.ShapeDtypeStruct((NC, HALF, DEGW), jnp.float32))
    scratch += [
        pltpu.VMEM((CHUNK, DEGW), jnp.float32),
        pltpu.VMEM((ZROWS, DEGW), jnp.float32),
        pltpu.VMEM_SHARED((HPAD, DEGW), jnp.float32),
    ]

  def body(table, src, dst, *rest):
    if with_deg:
      (out, deg_out, idx_s, idx_d, rows0, rows1, zb, sem0, sem1, agg_sh,
       ones_v, zbd, deg_sh) = rest
    else:
      out, idx_s, idx_d, rows0, rows1, zb, sem0, sem1, agg_sh = rest
    rows = rows0

    cid = lax.axis_index("c")
    sid = lax.axis_index("s")

    z0 = sid * RPZ
    for r in range(ZROWS):
      for c in range(F_IN // 16):
        zb[r, pl.ds(c * 16, 16)] = jnp.zeros((16,), jnp.float32)
    for j in range(RPZ // ZROWS):
      pltpu.sync_copy(zb, agg_sh.at[pl.ds(z0 + j * ZROWS, ZROWS)])
    if with_deg:
      for r in range(ZROWS):
        zbd[r, :] = jnp.zeros((DEGW,), jnp.float32)
      for r in range(CHUNK):
        ones_v[r, :] = jnp.ones((DEGW,), jnp.float32)
      for j in range(RPZ // ZROWS):
        pltpu.sync_copy(zbd, deg_sh.at[pl.ds(z0 + j * ZROWS, ZROWS)])
    plsc.subcore_barrier()

    base0 = sid * EPW
    lo = cid * HALF

    bufs = (rows0, rows1)
    sems = (sem0, sem1)
    trash = HALF + sid * 16 + jax.lax.iota(jnp.int32, 16)

    @pl.loop(0, NBLK)
    def _(blk):
      e0 = base0 + blk * BLK
      pltpu.sync_copy(src.at[pl.ds(e0, BLK)], idx_s)
      pltpu.sync_copy(dst.at[pl.ds(e0, BLK)], idx_d)
      for k in range(BLK // 16):
        sl = pl.ds(k * 16, 16)
        v = idx_d[sl] - lo
        ok = jnp.logical_and(v >= 0, v < HALF)
        idx_d[sl] = jnp.where(ok, v, trash)

      def gather(j):
        return pltpu.async_copy(
            table.at[idx_s.at[pl.ds(j * CHUNK, CHUNK)]],
            bufs[j % 2], sems[j % 2])

      cp = gather(0)
      for j in range(CPB):
        nxt = gather(j + 1) if j + 1 < CPB else None
        cp.wait()
        dv = idx_d.at[pl.ds(j * CHUNK, CHUNK)]
        pltpu.sync_copy(bufs[j % 2], agg_sh.at[dv], add=True)
        if with_deg:
          pltpu.sync_copy(ones_v, deg_sh.at[dv], add=True)
        cp = nxt

    plsc.subcore_barrier()

    r0 = sid * RPT
    copc = 80
    for j in range(RPT // copc):
      rs = pl.ds(r0 + j * copc, copc)
      pltpu.sync_copy(agg_sh.at[rs], rows.at[pl.ds(0, copc)])
      pltpu.sync_copy(rows.at[pl.ds(0, copc)], out.at[cid, rs])
      if with_deg:
        pltpu.sync_copy(deg_sh.at[rs], ones_v.at[pl.ds(0, copc)])
        pltpu.sync_copy(ones_v.at[pl.ds(0, copc)], deg_out.at[cid, rs])

  return pl.kernel(body, out_type=tuple(out_type), mesh=mesh,
                   scratch_types=scratch)


_sc_agg_deg = _make_sc_agg(True)
_sc_agg = _make_sc_agg(False)


ROWS_TC = 1000


def _rdeg(degs_ref):
  ones = jnp.ones((DEGW, F_IN), jnp.float32)
  d = jnp.dot(degs_ref[...], ones,
              preferred_element_type=jnp.float32) * (1.0 / DEGW)
  return 1.0 / jnp.maximum(d, 1.0)


def _tc1_body(parts_ref, degs_ref, x_ref, wl_ref, wr_ref, b_ref, out_ref):
  agg = parts_ref[...] * _rdeg(degs_ref)
  h = (jnp.dot(agg, wl_ref[...], preferred_element_type=jnp.float32)
       + jnp.dot(x_ref[...], wr_ref[...], preferred_element_type=jnp.float32)
       + b_ref[...])
  out_ref[...] = jnp.maximum(h, 0.0)


def _tc2_body(parts_ref, degs_ref, h_ref, wl_ref, wr_ref, b_ref, out_ref):
  agg = parts_ref[...] * _rdeg(degs_ref)
  z = (jnp.dot(agg, wl_ref[...], preferred_element_type=jnp.float32)
       + jnp.dot(h_ref[...], wr_ref[...], preferred_element_type=jnp.float32)
       + b_ref[...])
  m = jnp.max(z, axis=1, keepdims=True)
  e = z - m
  out_ref[...] = e - jnp.log(jnp.sum(jnp.exp(e), axis=1, keepdims=True))


def _tc_layer(body, dout, parts, degs, feat, wl, wr, b):
  grid = (N // ROWS_TC,)
  return pl.pallas_call(
      body,
      grid=grid,
      in_specs=[
          pl.BlockSpec((ROWS_TC, F_IN), lambda i: (i, 0)),
          pl.BlockSpec((ROWS_TC, DEGW), lambda i: (i, 0)),
          pl.BlockSpec((ROWS_TC, F_IN), lambda i: (i, 0)),
          pl.BlockSpec((F_IN, dout), lambda i: (0, 0)),
          pl.BlockSpec((F_IN, dout), lambda i: (0, 0)),
          pl.BlockSpec((1, dout), lambda i: (0, 0)),
      ],
      out_specs=pl.BlockSpec((ROWS_TC, dout), lambda i: (i, 0)),
      out_shape=jax.ShapeDtypeStruct((N, dout), jnp.float32),
  )(parts, degs, feat, wl, wr, b)


def kernel(x, edge_index, W1_l, W1_r, b1, W2_l, W2_r, b2):
  src = edge_index[0]
  dst = edge_index[1]
  parts1, degp = _sc_agg_deg(x, src, dst)
  parts1 = parts1.reshape(NC * HALF, F_IN)
  degf = degp.reshape(NC * HALF, DEGW)
  h = _tc_layer(_tc1_body, HID, parts1, degf, x, W1_l, W1_r,
                b1.reshape(1, HID))
  (parts2,) = _sc_agg(h, src, dst)
  out = _tc_layer(_tc2_body, CLS, parts2.reshape(NC * HALF, F_IN), degf, h,
                  W2_l, W2_r, b2.reshape(1, CLS))
  return out


# speedup vs baseline: 7.2692x; 1.0402x over previous
import jax
import jax.numpy as jnp
from jax import lax
from jax.experimental import pallas as pl
from jax.experimental.pallas import tpu as pltpu
from jax.experimental.pallas import tpu_sc as plsc

N = 10000
E = 320000
F_IN = 128
HID = 128
CLS = 64

NC = 2
NS = 16
HALF = 5120
HPAD = 5376
TRASH = HALF
EPW = E // NS
CHUNK = 80
BLK = 4000
CPB = BLK // CHUNK
NBLK = EPW // BLK
RPZ = HPAD // NS
RPT = HALF // NS
DEGW = 16
ZROWS = 16


def _make_sc_agg(with_deg):
  mesh = plsc.VectorSubcoreMesh(core_axis_name="c", subcore_axis_name="s")
  out_type = [jax.ShapeDtypeStruct((NC, HALF, F_IN), jnp.float32)]
  scratch = [
      pltpu.VMEM((BLK,), jnp.int32),
      pltpu.VMEM((BLK,), jnp.int32),
      pltpu.VMEM((CHUNK, F_IN), jnp.float32),
      pltpu.VMEM((CHUNK, F_IN), jnp.float32),
      pltpu.VMEM((ZROWS, F_IN), jnp.float32),
      pltpu.SemaphoreType.DMA,
      pltpu.SemaphoreType.DMA,
      pltpu.VMEM_SHARED((HPAD, F_IN), jnp.float32),
  ]
  if with_deg:
    out_type.append(jax.ShapeDtypeStruct((NC, HALF, DEGW), jnp.float32))
    scratch += [
        pltpu.VMEM((CHUNK, DEGW), jnp.float32),
        pltpu.VMEM((ZROWS, DEGW), jnp.float32),
        pltpu.VMEM_SHARED((HPAD, DEGW), jnp.float32),
    ]

  def body(table, src, dst, *rest):
    if with_deg:
      (out, deg_out, idx_s, idx_d, rows0, rows1, zb, sem0, sem1, agg_sh,
       ones_v, zbd, deg_sh) = rest
    else:
      out, idx_s, idx_d, rows0, rows1, zb, sem0, sem1, agg_sh = rest
    rows = rows0

    cid = lax.axis_index("c")
    sid = lax.axis_index("s")

    z0 = sid * RPZ
    for r in range(ZROWS):
      for c in range(F_IN // 16):
        zb[r, pl.ds(c * 16, 16)] = jnp.zeros((16,), jnp.float32)
    for j in range(RPZ // ZROWS):
      pltpu.sync_copy(zb, agg_sh.at[pl.ds(z0 + j * ZROWS, ZROWS)])
    if with_deg:
      for r in range(ZROWS):
        zbd[r, :] = jnp.zeros((DEGW,), jnp.float32)
      for r in range(CHUNK):
        ones_v[r, :] = jnp.ones((DEGW,), jnp.float32)
      for j in range(RPZ // ZROWS):
        pltpu.sync_copy(zbd, deg_sh.at[pl.ds(z0 + j * ZROWS, ZROWS)])
    plsc.subcore_barrier()

    base0 = sid * EPW
    lo = cid * HALF

    bufs = (rows0, rows1)
    sems = (sem0, sem1)
    trash = HALF + sid * 16 + jax.lax.iota(jnp.int32, 16)

    @pl.loop(0, NBLK)
    def _(blk):
      e0 = base0 + blk * BLK
      pltpu.sync_copy(src.at[pl.ds(e0, BLK)], idx_s)
      pltpu.sync_copy(dst.at[pl.ds(e0, BLK)], idx_d)
      for k in range(BLK // 16):
        sl = pl.ds(k * 16, 16)
        v = idx_d[sl] - lo
        ok = jnp.logical_and(v >= 0, v < HALF)
        idx_d[sl] = jnp.where(ok, v, trash)

      def gather(j):
        return pltpu.async_copy(
            table.at[idx_s.at[pl.ds(j * CHUNK, CHUNK)]],
            bufs[j % 2], sems[j % 2])

      cp = gather(0)
      for j in range(CPB):
        nxt = gather(j + 1) if j + 1 < CPB else None
        cp.wait()
        dv = idx_d.at[pl.ds(j * CHUNK, CHUNK)]
        pltpu.sync_copy(bufs[j % 2], agg_sh.at[dv], add=True)
        if with_deg:
          pltpu.sync_copy(ones_v, deg_sh.at[dv], add=True)
        cp = nxt

    plsc.subcore_barrier()

    r0 = sid * RPT
    copc = 80
    for j in range(RPT // copc):
      rs = pl.ds(r0 + j * copc, copc)
      pltpu.sync_copy(agg_sh.at[rs], rows.at[pl.ds(0, copc)])
      pltpu.sync_copy(rows.at[pl.ds(0, copc)], out.at[cid, rs])
      if with_deg:
        pltpu.sync_copy(deg_sh.at[rs], ones_v.at[pl.ds(0, copc)])
        pltpu.sync_copy(ones_v.at[pl.ds(0, copc)], deg_out.at[cid, rs])

  return pl.kernel(body, out_type=tuple(out_type), mesh=mesh,
                   scratch_types=scratch)


_sc_agg_deg = _make_sc_agg(True)
_sc_agg = _make_sc_agg(False)


ROWS_TC = 1000


def _rdeg(degs_ref):
  ones = jnp.ones((DEGW, F_IN), jnp.float32)
  d = jnp.dot(degs_ref[...], ones,
              preferred_element_type=jnp.float32) * (1.0 / DEGW)
  return 1.0 / jnp.maximum(d, 1.0)


def _tc1_body(parts_ref, degs_ref, x_ref, wl_ref, wr_ref, b_ref, out_ref):
  agg = parts_ref[...] * _rdeg(degs_ref)
  h = (jnp.dot(agg, wl_ref[...], preferred_element_type=jnp.float32)
       + jnp.dot(x_ref[...], wr_ref[...], preferred_element_type=jnp.float32)
       + b_ref[...])
  out_ref[...] = jnp.maximum(h, 0.0)


def _tc2_body(parts_ref, degs_ref, h_ref, wl_ref, wr_ref, b_ref, out_ref):
  agg = parts_ref[...] * _rdeg(degs_ref)
  z = (jnp.dot(agg, wl_ref[...], preferred_element_type=jnp.float32)
       + jnp.dot(h_ref[...], wr_ref[...], preferred_element_type=jnp.float32)
       + b_ref[...])
  m = jnp.max(z, axis=1, keepdims=True)
  e = z - m
  out_ref[...] = e - jnp.log(jnp.sum(jnp.exp(e), axis=1, keepdims=True))


def _tc_layer(body, dout, parts, degs, feat, wl, wr, b):
  grid = (N // ROWS_TC,)
  return pl.pallas_call(
      body,
      grid=grid,
      in_specs=[
          pl.BlockSpec((ROWS_TC, F_IN), lambda i: (i, 0)),
          pl.BlockSpec((ROWS_TC, DEGW), lambda i: (i, 0)),
          pl.BlockSpec((ROWS_TC, F_IN), lambda i: (i, 0)),
          pl.BlockSpec((F_IN, dout), lambda i: (0, 0)),
          pl.BlockSpec((F_IN, dout), lambda i: (0, 0)),
          pl.BlockSpec((1, dout), lambda i: (0, 0)),
      ],
      out_specs=pl.BlockSpec((ROWS_TC, dout), lambda i: (i, 0)),
      out_shape=jax.ShapeDtypeStruct((N, dout), jnp.float32),
  )(parts, degs, feat, wl, wr, b)


def kernel(x, edge_index, W1_l, W1_r, b1, W2_l, W2_r, b2):
  src = edge_index[0]
  dst = edge_index[1]
  parts1, degp = _sc_agg_deg(x, src, dst)
  parts1 = parts1.reshape(NC * HALF, F_IN)
  degf = degp.reshape(NC * HALF, DEGW)
  h = _tc_layer(_tc1_body, HID, parts1, degf, x, W1_l, W1_r,
                b1.reshape(1, HID))
  (parts2,) = _sc_agg(h, src, dst)
  out = _tc_layer(_tc2_body, CLS, parts2.reshape(NC * HALF, F_IN), degf, h,
                  W2_l, W2_r, b2.reshape(1, CLS))
  return out
